# baseline (speedup 1.0000x reference)
import functools

import jax
import jax.numpy as jnp
from jax import lax
from jax.experimental import pallas as pl
from jax.experimental.pallas import tpu as pltpu
from jax.experimental.pallas import tpu_sc as plsc

K = 512
N = 10000
EG = 150000
B = 128
H = 50

NC, NS, LANES = 2, 16, 16
NW = NC * NS
EB = 128
EPAD = 163840
NB = EPAD // NS // EB
NPAD = 10240
RPT = NPAD // NS
CHUNKS = 4
CW = 128
DEGW = 128

ROW_BLK = 400

@functools.cache
def _sc_mesh():
    return plsc.VectorSubcoreMesh(core_axis_name="c", subcore_axis_name="s",
                                  num_cores=NC, num_subcores=NS)


def _fill(ref, rows, width, val):
    def body(r, _):
        for k in range(width // LANES):
            ref[r, pl.ds(k * LANES, LANES)] = jnp.full((LANES,), val,
                                                       jnp.float32)
        return 0
    lax.fori_loop(0, rows, body, 0)


IG = 8


def _sc_propagate_body(*refs):
    (h0, h1, h2, h3, edges, agg_out,
     acc_sh, idxs, idxd, rows0, rows1, zero_v,
     gsem0, gsem1, ssem0, ssem1) = refs
    tables = (h0, h1, h2, h3)
    rows = (rows0, rows1)
    gsem = (gsem0, gsem1)
    ssem = (ssem0, ssem1)
    cid = lax.axis_index("c")
    sid = lax.axis_index("s")
    r0 = pl.multiple_of(sid * RPT, 32)
    b0 = pl.multiple_of(sid * NB, 8)

    _fill(zero_v, 32, CW, 0.0)

    def wait_bytes(sem, dst):
        pltpu.make_async_copy(tables[0].at[pl.ds(0, dst.shape[0])],
                              dst, sem).wait()

    def zbody(j, _):
        pltpu.sync_copy(zero_v, acc_sh.at[pl.ds(r0 + j * 32, 32)])
        return 0

    lax.fori_loop(0, RPT // 32, zbody, 0)
    plsc.subcore_barrier()

    for g in range(3):
        for c in range(CHUNKS):
            owner = c // 2

            @pl.when(cid == owner)
            def _pass(g=g, c=c):
                tbl = tables[c]

                def body(i, _):
                    nb = pl.multiple_of(b0 + i * IG, 8)
                    pltpu.sync_copy(edges.at[g, 0, pl.ds(nb, IG)], idxs)
                    pltpu.sync_copy(edges.at[g, 1, pl.ds(nb, IG)], idxd)
                    for k in range(IG):
                        s = k % 2
                        pltpu.async_copy(tbl.at[idxs.at[k]], rows[s],
                                         gsem[s]).wait()
                        pltpu.sync_copy(rows[s], acc_sh.at[idxd.at[k]],
                                        add=True)
                    return 0

                lax.fori_loop(0, NB // IG, body, 0)
                plsc.subcore_barrier()

                for j in range(RPT // EB):
                    rr = r0 + j * EB
                    s = j % 2
                    if j >= 2:
                        wait_bytes(ssem[s], rows[s])
                    pltpu.sync_copy(acc_sh.at[pl.ds(rr, EB)], rows[s])
                    for z in range(EB // 32):
                        pltpu.sync_copy(zero_v,
                                        acc_sh.at[pl.ds(rr + z * 32, 32)])
                    pltpu.async_copy(rows[s],
                                     agg_out.at[c, g, pl.ds(rr, EB)],
                                     ssem[s])
                wait_bytes(ssem[(RPT // EB - 2) % 2], rows[0])
                wait_bytes(ssem[(RPT // EB - 1) % 2], rows[1])
                plsc.subcore_barrier()


def _sc_propagate(h_chunks, edges):
    fn = pl.kernel(
        _sc_propagate_body,
        out_type=jax.ShapeDtypeStruct((CHUNKS, 3, NPAD, CW), jnp.float32),
        mesh=_sc_mesh(),
        scratch_types=[
            pltpu.VMEM_SHARED((NPAD, CW), jnp.float32),
            pltpu.VMEM((IG, EB), jnp.int32),
            pltpu.VMEM((IG, EB), jnp.int32),
            pltpu.VMEM((EB, CW), jnp.float32),
            pltpu.VMEM((EB, CW), jnp.float32),
            pltpu.VMEM((32, CW), jnp.float32),
            pltpu.SemaphoreType.DMA,
            pltpu.SemaphoreType.DMA,
            pltpu.SemaphoreType.DMA,
            pltpu.SemaphoreType.DMA,
        ],
    )
    return fn(*h_chunks, edges)


def _sc_deg_body(edges, deg_out, deg_sh, idx_d, ones_v, zero16, deg_buf,
                 sem):
    cid = lax.axis_index("c")
    sid = lax.axis_index("s")
    r0 = pl.multiple_of(sid * RPT, 32)
    b0 = pl.multiple_of(sid * NB, 8)

    _fill(ones_v, EB, DEGW, 1.0)
    _fill(zero16, 32, DEGW, 0.0)

    for g in range(3):
        owner = 0 if g < 2 else 1

        @pl.when(cid == owner)
        def _pass(g=g):
            def zbody(j, _):
                pltpu.sync_copy(zero16, deg_sh.at[pl.ds(r0 + j * 32, 32)])
                return 0

            lax.fori_loop(0, RPT // 32, zbody, 0)
            plsc.subcore_barrier()

            def body(grp, _):
                pltpu.sync_copy(
                    edges.at[g, 1, pl.ds(pl.multiple_of(b0 + grp * 8, 8),
                                         8)],
                    idx_d)
                for j in range(8):
                    pltpu.sync_copy(ones_v, deg_sh.at[idx_d.at[j]],
                                    add=True)
                return 0

            lax.fori_loop(0, NB // 8, body, 0)
            plsc.subcore_barrier()

            for j in range(RPT // EB):
                rr = r0 + j * EB
                pltpu.sync_copy(deg_sh.at[pl.ds(rr, EB)], deg_buf)
                pltpu.sync_copy(deg_buf, deg_out.at[g, pl.ds(rr, EB)])


def _sc_deg(edges):
    fn = pl.kernel(
        _sc_deg_body,
        out_type=jax.ShapeDtypeStruct((3, NPAD, DEGW), jnp.float32),
        mesh=_sc_mesh(),
        scratch_types=[
            pltpu.VMEM_SHARED((NPAD, DEGW), jnp.float32),
            pltpu.VMEM((8, EB), jnp.int32),
            pltpu.VMEM((EB, DEGW), jnp.float32),
            pltpu.VMEM((32, DEGW), jnp.float32),
            pltpu.VMEM((EB, DEGW), jnp.float32),
            pltpu.SemaphoreType.DMA,
        ],
    )
    return fn(edges)


def _sc_final_gather_body(*refs):
    (tabs, hist_idx, exer_idx,
     hist2, hist3, x0, x1, x2, x3,
     hidx128, hidx72, eidx, rows_v, rows72, rows8, sem) = (
        refs[:16], refs[16], refs[17],
        refs[18], refs[19], refs[20], refs[21], refs[22], refs[23],
        refs[24], refs[25], refs[26], refs[27], refs[28], refs[29],
        refs[30])
    cid = lax.axis_index("c")
    sid = lax.axis_index("s")
    wid = sid * NC + cid
    hb = wid * 200

    pltpu.sync_copy(hist_idx.at[pl.ds(hb, 128)], hidx128)
    pltpu.sync_copy(hist_idx.at[pl.ds(hb + 128, 72)], hidx72)
    for li, out in ((2, hist2), (3, hist3)):
        for c in range(CHUNKS):
            tab = tabs[li * CHUNKS + c]
            pltpu.async_copy(tab.at[hidx128], rows_v, sem).wait()
            pltpu.sync_copy(rows_v, out.at[c, pl.ds(hb, 128)])
            pltpu.async_copy(tab.at[hidx72], rows72, sem).wait()
            pltpu.sync_copy(rows72, out.at[c, pl.ds(hb + 128, 72)])

    eb = wid * 8
    pltpu.sync_copy(exer_idx.at[pl.ds(eb, 8)], eidx)
    for li, out in ((0, x0), (1, x1), (2, x2), (3, x3)):
        for c in range(CHUNKS):
            tab = tabs[li * CHUNKS + c]
            pltpu.async_copy(tab.at[eidx], rows8, sem).wait()
            pltpu.sync_copy(rows8, out.at[c, pl.ds(eb, 8)])


def _sc_final_gather(all_chunks, hist_idx, exer_idx):
    outs = (
        jax.ShapeDtypeStruct((CHUNKS, 6400, CW), jnp.float32),
        jax.ShapeDtypeStruct((CHUNKS, 6400, CW), jnp.float32),
        jax.ShapeDtypeStruct((CHUNKS, 256, CW), jnp.float32),
        jax.ShapeDtypeStruct((CHUNKS, 256, CW), jnp.float32),
        jax.ShapeDtypeStruct((CHUNKS, 256, CW), jnp.float32),
        jax.ShapeDtypeStruct((CHUNKS, 256, CW), jnp.float32),
    )
    fn = pl.kernel(
        _sc_final_gather_body,
        out_type=outs,
        mesh=_sc_mesh(),
        scratch_types=[
            pltpu.VMEM((128,), jnp.int32),
            pltpu.VMEM((72,), jnp.int32),
            pltpu.VMEM((8,), jnp.int32),
            pltpu.VMEM((128, CW), jnp.float32),
            pltpu.VMEM((72, CW), jnp.float32),
            pltpu.VMEM((8, CW), jnp.float32),
            pltpu.SemaphoreType.DMA,
        ],
    )
    return fn(*all_chunks, hist_idx, exer_idx)


def _chunk_body(x_ref, o0, o1, o2, o3):
    for c, o in enumerate((o0, o1, o2, o3)):
        o[...] = x_ref[:, c * CW:(c + 1) * CW]


def _chunk_split(x):
    grid = N // ROW_BLK
    return pl.pallas_call(
        _chunk_body,
        grid=(grid,),
        in_specs=[pl.BlockSpec((ROW_BLK, K), lambda i: (i, 0))],
        out_specs=[pl.BlockSpec((ROW_BLK, CW), lambda i: (i, 0))] * CHUNKS,
        out_shape=[jax.ShapeDtypeStruct((N, CW), jnp.float32)] * CHUNKS,
    )(x)


def _layer_mm_body(agg_ref, deg_ref, w_ref, bsum_ref, o0, o1, o2, o3):
    acc = jnp.zeros((ROW_BLK, K), jnp.float32)
    for g in range(3):
        rd = 1.0 / jnp.maximum(deg_ref[g, :, 0:1], 1.0)
        for c in range(CHUNKS):
            x = agg_ref[c, g] * rd
            acc = acc + jnp.dot(x, w_ref[g, c * CW:(c + 1) * CW, :],
                                preferred_element_type=jnp.float32)
    acc = acc + bsum_ref[...]
    for c, o in enumerate((o0, o1, o2, o3)):
        o[...] = acc[:, c * CW:(c + 1) * CW]


def _layer_matmul(agg, deg, w3, bsum):
    grid = N // ROW_BLK
    return pl.pallas_call(
        _layer_mm_body,
        grid=(grid,),
        in_specs=[
            pl.BlockSpec((CHUNKS, 3, ROW_BLK, CW), lambda i: (0, 0, i, 0)),
            pl.BlockSpec((3, ROW_BLK, DEGW), lambda i: (0, i, 0)),
            pl.BlockSpec((3, K, K), lambda i: (0, 0, 0)),
            pl.BlockSpec((1, K), lambda i: (0, 0)),
        ],
        out_specs=[pl.BlockSpec((ROW_BLK, CW), lambda i: (i, 0))] * CHUNKS,
        out_shape=[jax.ShapeDtypeStruct((N, CW), jnp.float32)] * CHUNKS,
    )(agg, deg, w3, bsum)


def _pool_body(h2_ref, h3_ref, out_ref):
    for c in range(CHUNKS):
        s = (h2_ref[c] + h3_ref[c]).reshape(16, H, CW)
        out_ref[:, c * CW:(c + 1) * CW] = jnp.sum(s, axis=1) * (0.5 / H)


def _pool(hist2, hist3):
    grid = B // 16
    return pl.pallas_call(
        _pool_body,
        grid=(grid,),
        in_specs=[pl.BlockSpec((CHUNKS, 16 * H, CW), lambda i: (0, i, 0))] * 2,
        out_specs=pl.BlockSpec((16, K), lambda i: (i, 0)),
        out_shape=jax.ShapeDtypeStruct((B, K), jnp.float32),
    )(hist2, hist3)


def _final_body(stu_ref, kn_ref, edisc_ref,
                c0, c1, c2, c3, x0, x1, x2, x3,
                wfs_ref, bfs_ref, wfe_ref, bfe_ref, wp3_ref, bp3_ref,
                out_ref):
    conc_c = [(c0[c] + c1[c] + c2[c] + c3[c]) * 0.25 for c in range(CHUNKS)]
    concept = jnp.concatenate(conc_c, axis=1)
    exer_c = [(x0[c] + x1[c] + x2[c] + x3[c])[0:B] * 0.25
              for c in range(CHUNKS)]
    exer = jnp.concatenate(exer_c, axis=1)

    wa = wfs_ref[0, :K]
    wb = wfs_ref[0, K:]
    wea = wfe_ref[0, :K]
    web = wfe_ref[0, K:]
    s = jnp.sum(stu_ref[...] * wa[None, :], axis=1, keepdims=True)
    t = jnp.sum(concept * wb[None, :], axis=1, keepdims=True)
    u = jnp.sum(exer * wea[None, :], axis=1, keepdims=True)
    v = jnp.sum(concept * web[None, :], axis=1, keepdims=True)
    prof = jax.nn.sigmoid(s + t[:, 0][None, :] + bfs_ref[0])
    kdiff = jax.nn.sigmoid(u + v[:, 0][None, :] + bfe_ref[0])
    edisc = jax.nn.sigmoid(edisc_ref[...]) * 10.0
    input_x = edisc * (prof - kdiff) * kn_ref[...]
    out_ref[...] = jax.nn.sigmoid(
        jnp.sum(input_x * wp3_ref[0][None, :], axis=1, keepdims=True)
        + bp3_ref[0])


def _final_stage(stu_emb, kn_emb, edisc_rows, conc_stacks, exer_rows,
                 w_full_stu, b_full_stu, w_full_exer, b_full_exer, w_p3, b_p3):
    return pl.pallas_call(
        _final_body,
        out_shape=jax.ShapeDtypeStruct((B, 1), jnp.float32),
    )(stu_emb, kn_emb, edisc_rows, *conc_stacks, *exer_rows,
      w_full_stu, b_full_stu.reshape(1, 1), w_full_exer,
      b_full_exer.reshape(1, 1), w_p3, b_p3.reshape(1, 1))


def _prep_edges(ei):
    pad = EPAD - EG
    tail = jnp.stack([jnp.zeros((pad,), jnp.int32),
                      jnp.full((pad,), N, jnp.int32)])
    return jnp.concatenate([ei.astype(jnp.int32), tail], axis=1)


def kernel(stu_id, exer_id, kn_emb, input_knowedge_ids, history,
           ei_sim, ei_pre, ei_exc,
           entity, e_disc_table, W_gcn, b_gcn,
           W_full_stu, b_full_stu, W_full_exer, b_full_exer, W_p3, b_p3):
    edges = jnp.stack([_prep_edges(ei_sim), _prep_edges(ei_pre),
                       _prep_edges(ei_exc)]
                      ).reshape(3, 2, EPAD // EB, EB)

    e0c = _chunk_split(entity)
    deg = _sc_deg(edges)
    embs_c = [e0c]
    h_c = e0c
    for l in range(3):
        agg = _sc_propagate(h_c, edges)
        w3 = W_gcn[3 * l:3 * l + 3]
        bsum = jnp.sum(b_gcn[3 * l:3 * l + 3], axis=0, keepdims=True)
        h_c = _layer_matmul(agg, deg, w3, bsum)
        embs_c.append(h_c)

    all_chunks = [t for lc in embs_c for t in lc]
    hist_idx = (history.astype(jnp.int32) + K).reshape(-1)
    exer_idx = jnp.concatenate([exer_id.astype(jnp.int32) + K,
                                jnp.full((256 - B,), K, jnp.int32)])
    hist2, hist3, x0, x1, x2, x3 = _sc_final_gather(
        all_chunks, hist_idx, exer_idx)

    stu_emb = _pool(hist2, hist3)
    conc_stacks = [jnp.stack([c[:K] for c in lc]) for lc in embs_c]
    edisc_rows = e_disc_table[exer_id]
    return _final_stage(stu_emb, kn_emb, edisc_rows, conc_stacks,
                        (x0, x1, x2, x3),
                        W_full_stu, b_full_stu, W_full_exer, b_full_exer,
                        W_p3, b_p3)


# speedup vs baseline: 2.7615x; 2.7615x over previous
import functools

import jax
import jax.numpy as jnp
from jax import lax
from jax.experimental import pallas as pl
from jax.experimental.pallas import tpu as pltpu
from jax.experimental.pallas import tpu_sc as plsc

K = 512
N = 10000
EG = 150000
B = 128
H = 50

NC, NS, LANES = 2, 16, 16
NW = NC * NS
EB = 128
EPAD = 151552
NB = EPAD // NS // EB
NPAD = 10240
RPT = NPAD // NS
CHUNKS = 4
CW = 128
DEGW = 128

ROW_BLK = 400

@functools.cache
def _sc_mesh():
    return plsc.VectorSubcoreMesh(core_axis_name="c", subcore_axis_name="s",
                                  num_cores=NC, num_subcores=NS)


def _fill(ref, rows, width, val):
    def body(r, _):
        for k in range(width // LANES):
            ref[r, pl.ds(k * LANES, LANES)] = jnp.full((LANES,), val,
                                                       jnp.float32)
        return 0
    lax.fori_loop(0, rows, body, 0)


IG = 8


def _sc_propagate_body(*refs):
    (h0, h1, h2, h3, edges, agg_out,
     acc_sh, idxs, idxd, rows0, rows1, zero_v,
     gsem0, gsem1, ssem0, ssem1) = refs
    tables = (h0, h1, h2, h3)
    rows = (rows0, rows1)
    gsem = (gsem0, gsem1)
    ssem = (ssem0, ssem1)
    cid = lax.axis_index("c")
    sid = lax.axis_index("s")
    r0 = pl.multiple_of(sid * RPT, 32)
    e0 = pl.multiple_of(sid * (EPAD // NS), 8)

    _fill(zero_v, 32, CW, 0.0)

    def wait_bytes(sem, dst):
        pltpu.make_async_copy(tables[0].at[pl.ds(0, dst.shape[0])],
                              dst, sem).wait()

    def zbody(j, _):
        pltpu.sync_copy(zero_v, acc_sh.at[pl.ds(r0 + j * 32, 32)])
        return 0

    lax.fori_loop(0, RPT // 32, zbody, 0)
    plsc.subcore_barrier()

    for g in range(3):
        for c in range(CHUNKS):
            owner = c // 2

            @pl.when(cid == owner)
            def _pass(g=g, c=c):
                tbl = tables[c]

                def body(b, _):
                    off = e0 + b * EB
                    pltpu.sync_copy(edges.at[g, 0, pl.ds(off, EB)], idxs)
                    pltpu.sync_copy(edges.at[g, 1, pl.ds(off, EB)], idxd)
                    pltpu.async_copy(tbl.at[idxs], rows[0], gsem[0]).wait()
                    pltpu.sync_copy(rows[0], acc_sh.at[idxd], add=True)
                    return 0

                lax.fori_loop(0, NB, body, 0)
                plsc.subcore_barrier()

                for j in range(RPT // EB):
                    rr = r0 + j * EB
                    s = j % 2
                    if j >= 2:
                        wait_bytes(ssem[s], rows[s])
                    pltpu.sync_copy(acc_sh.at[pl.ds(rr, EB)], rows[s])
                    for z in range(EB // 32):
                        pltpu.sync_copy(zero_v,
                                        acc_sh.at[pl.ds(rr + z * 32, 32)])
                    pltpu.async_copy(rows[s],
                                     agg_out.at[c, g, pl.ds(rr, EB)],
                                     ssem[s])
                wait_bytes(ssem[(RPT // EB - 2) % 2], rows[0])
                wait_bytes(ssem[(RPT // EB - 1) % 2], rows[1])
                plsc.subcore_barrier()


def _sc_propagate(h_chunks, edges):
    fn = pl.kernel(
        _sc_propagate_body,
        out_type=jax.ShapeDtypeStruct((CHUNKS, 3, NPAD, CW), jnp.float32),
        mesh=_sc_mesh(),
        scratch_types=[
            pltpu.VMEM_SHARED((NPAD, CW), jnp.float32),
            pltpu.VMEM((EB,), jnp.int32),
            pltpu.VMEM((EB,), jnp.int32),
            pltpu.VMEM((EB, CW), jnp.float32),
            pltpu.VMEM((EB, CW), jnp.float32),
            pltpu.VMEM((32, CW), jnp.float32),
            pltpu.SemaphoreType.DMA,
            pltpu.SemaphoreType.DMA,
            pltpu.SemaphoreType.DMA,
            pltpu.SemaphoreType.DMA,
        ],
    )
    return fn(*h_chunks, edges)


def _sc_deg_body(edges, deg_out, deg_sh, idx_d, ones_v, zero16, deg_buf,
                 sem):
    cid = lax.axis_index("c")
    sid = lax.axis_index("s")
    r0 = pl.multiple_of(sid * RPT, 32)
    e0 = pl.multiple_of(sid * (EPAD // NS), 8)

    _fill(ones_v, EB, DEGW, 1.0)
    _fill(zero16, 32, DEGW, 0.0)

    for g in range(3):
        owner = 0 if g < 2 else 1

        @pl.when(cid == owner)
        def _pass(g=g):
            def zbody(j, _):
                pltpu.sync_copy(zero16, deg_sh.at[pl.ds(r0 + j * 32, 32)])
                return 0

            lax.fori_loop(0, RPT // 32, zbody, 0)
            plsc.subcore_barrier()

            def body(b, _):
                off = e0 + b * EB
                pltpu.sync_copy(edges.at[g, 1, pl.ds(off, EB)], idx_d)
                pltpu.sync_copy(ones_v, deg_sh.at[idx_d], add=True)
                return 0

            lax.fori_loop(0, NB, body, 0)
            plsc.subcore_barrier()

            for j in range(RPT // EB):
                rr = r0 + j * EB
                pltpu.sync_copy(deg_sh.at[pl.ds(rr, EB)], deg_buf)
                pltpu.sync_copy(deg_buf, deg_out.at[g, pl.ds(rr, EB)])


def _sc_deg(edges):
    fn = pl.kernel(
        _sc_deg_body,
        out_type=jax.ShapeDtypeStruct((3, NPAD, DEGW), jnp.float32),
        mesh=_sc_mesh(),
        scratch_types=[
            pltpu.VMEM_SHARED((NPAD, DEGW), jnp.float32),
            pltpu.VMEM((EB,), jnp.int32),
            pltpu.VMEM((EB, DEGW), jnp.float32),
            pltpu.VMEM((32, DEGW), jnp.float32),
            pltpu.VMEM((EB, DEGW), jnp.float32),
            pltpu.SemaphoreType.DMA,
        ],
    )
    return fn(edges)


def _sc_final_gather_body(*refs):
    (tabs, hist_idx, exer_idx,
     hist2, hist3, x0, x1, x2, x3,
     hidx128, hidx72, eidx, rows_v, rows72, rows8, sem) = (
        refs[:16], refs[16], refs[17],
        refs[18], refs[19], refs[20], refs[21], refs[22], refs[23],
        refs[24], refs[25], refs[26], refs[27], refs[28], refs[29],
        refs[30])
    cid = lax.axis_index("c")
    sid = lax.axis_index("s")
    wid = sid * NC + cid
    hb = wid * 200

    pltpu.sync_copy(hist_idx.at[pl.ds(hb, 128)], hidx128)
    pltpu.sync_copy(hist_idx.at[pl.ds(hb + 128, 72)], hidx72)
    for li, out in ((2, hist2), (3, hist3)):
        for c in range(CHUNKS):
            tab = tabs[li * CHUNKS + c]
            pltpu.async_copy(tab.at[hidx128], rows_v, sem).wait()
            pltpu.sync_copy(rows_v, out.at[c, pl.ds(hb, 128)])
            pltpu.async_copy(tab.at[hidx72], rows72, sem).wait()
            pltpu.sync_copy(rows72, out.at[c, pl.ds(hb + 128, 72)])

    eb = wid * 8
    pltpu.sync_copy(exer_idx.at[pl.ds(eb, 8)], eidx)
    for li, out in ((0, x0), (1, x1), (2, x2), (3, x3)):
        for c in range(CHUNKS):
            tab = tabs[li * CHUNKS + c]
            pltpu.async_copy(tab.at[eidx], rows8, sem).wait()
            pltpu.sync_copy(rows8, out.at[c, pl.ds(eb, 8)])


def _sc_final_gather(all_chunks, hist_idx, exer_idx):
    outs = (
        jax.ShapeDtypeStruct((CHUNKS, 6400, CW), jnp.float32),
        jax.ShapeDtypeStruct((CHUNKS, 6400, CW), jnp.float32),
        jax.ShapeDtypeStruct((CHUNKS, 256, CW), jnp.float32),
        jax.ShapeDtypeStruct((CHUNKS, 256, CW), jnp.float32),
        jax.ShapeDtypeStruct((CHUNKS, 256, CW), jnp.float32),
        jax.ShapeDtypeStruct((CHUNKS, 256, CW), jnp.float32),
    )
    fn = pl.kernel(
        _sc_final_gather_body,
        out_type=outs,
        mesh=_sc_mesh(),
        scratch_types=[
            pltpu.VMEM((128,), jnp.int32),
            pltpu.VMEM((72,), jnp.int32),
            pltpu.VMEM((8,), jnp.int32),
            pltpu.VMEM((128, CW), jnp.float32),
            pltpu.VMEM((72, CW), jnp.float32),
            pltpu.VMEM((8, CW), jnp.float32),
            pltpu.SemaphoreType.DMA,
        ],
    )
    return fn(*all_chunks, hist_idx, exer_idx)


def _chunk_body(x_ref, o0, o1, o2, o3):
    for c, o in enumerate((o0, o1, o2, o3)):
        o[...] = x_ref[:, c * CW:(c + 1) * CW]


def _chunk_split(x):
    grid = N // ROW_BLK
    return pl.pallas_call(
        _chunk_body,
        grid=(grid,),
        in_specs=[pl.BlockSpec((ROW_BLK, K), lambda i: (i, 0))],
        out_specs=[pl.BlockSpec((ROW_BLK, CW), lambda i: (i, 0))] * CHUNKS,
        out_shape=[jax.ShapeDtypeStruct((N, CW), jnp.float32)] * CHUNKS,
    )(x)


def _layer_mm_body(agg_ref, deg_ref, w_ref, bsum_ref, o0, o1, o2, o3):
    acc = jnp.zeros((ROW_BLK, K), jnp.float32)
    for g in range(3):
        rd = 1.0 / jnp.maximum(deg_ref[g, :, 0:1], 1.0)
        for c in range(CHUNKS):
            x = agg_ref[c, g] * rd
            acc = acc + jnp.dot(x, w_ref[g, c * CW:(c + 1) * CW, :],
                                preferred_element_type=jnp.float32)
    acc = acc + bsum_ref[...]
    for c, o in enumerate((o0, o1, o2, o3)):
        o[...] = acc[:, c * CW:(c + 1) * CW]


def _layer_matmul(agg, deg, w3, bsum):
    grid = N // ROW_BLK
    return pl.pallas_call(
        _layer_mm_body,
        grid=(grid,),
        in_specs=[
            pl.BlockSpec((CHUNKS, 3, ROW_BLK, CW), lambda i: (0, 0, i, 0)),
            pl.BlockSpec((3, ROW_BLK, DEGW), lambda i: (0, i, 0)),
            pl.BlockSpec((3, K, K), lambda i: (0, 0, 0)),
            pl.BlockSpec((1, K), lambda i: (0, 0)),
        ],
        out_specs=[pl.BlockSpec((ROW_BLK, CW), lambda i: (i, 0))] * CHUNKS,
        out_shape=[jax.ShapeDtypeStruct((N, CW), jnp.float32)] * CHUNKS,
    )(agg, deg, w3, bsum)


def _pool_body(h2_ref, h3_ref, out_ref):
    for c in range(CHUNKS):
        s = (h2_ref[c] + h3_ref[c]).reshape(16, H, CW)
        out_ref[:, c * CW:(c + 1) * CW] = jnp.sum(s, axis=1) * (0.5 / H)


def _pool(hist2, hist3):
    grid = B // 16
    return pl.pallas_call(
        _pool_body,
        grid=(grid,),
        in_specs=[pl.BlockSpec((CHUNKS, 16 * H, CW), lambda i: (0, i, 0))] * 2,
        out_specs=pl.BlockSpec((16, K), lambda i: (i, 0)),
        out_shape=jax.ShapeDtypeStruct((B, K), jnp.float32),
    )(hist2, hist3)


def _final_body(stu_ref, kn_ref, edisc_ref,
                c0, c1, c2, c3, x0, x1, x2, x3,
                wfs_ref, bfs_ref, wfe_ref, bfe_ref, wp3_ref, bp3_ref,
                out_ref):
    conc_c = [(c0[c] + c1[c] + c2[c] + c3[c]) * 0.25 for c in range(CHUNKS)]
    concept = jnp.concatenate(conc_c, axis=1)
    exer_c = [(x0[c] + x1[c] + x2[c] + x3[c])[0:B] * 0.25
              for c in range(CHUNKS)]
    exer = jnp.concatenate(exer_c, axis=1)

    wa = wfs_ref[0, :K]
    wb = wfs_ref[0, K:]
    wea = wfe_ref[0, :K]
    web = wfe_ref[0, K:]
    s = jnp.sum(stu_ref[...] * wa[None, :], axis=1, keepdims=True)
    t = jnp.sum(concept * wb[None, :], axis=1, keepdims=True)
    u = jnp.sum(exer * wea[None, :], axis=1, keepdims=True)
    v = jnp.sum(concept * web[None, :], axis=1, keepdims=True)
    prof = jax.nn.sigmoid(s + t[:, 0][None, :] + bfs_ref[0])
    kdiff = jax.nn.sigmoid(u + v[:, 0][None, :] + bfe_ref[0])
    edisc = jax.nn.sigmoid(edisc_ref[...]) * 10.0
    input_x = edisc * (prof - kdiff) * kn_ref[...]
    out_ref[...] = jax.nn.sigmoid(
        jnp.sum(input_x * wp3_ref[0][None, :], axis=1, keepdims=True)
        + bp3_ref[0])


def _final_stage(stu_emb, kn_emb, edisc_rows, conc_stacks, exer_rows,
                 w_full_stu, b_full_stu, w_full_exer, b_full_exer, w_p3, b_p3):
    return pl.pallas_call(
        _final_body,
        out_shape=jax.ShapeDtypeStruct((B, 1), jnp.float32),
    )(stu_emb, kn_emb, edisc_rows, *conc_stacks, *exer_rows,
      w_full_stu, b_full_stu.reshape(1, 1), w_full_exer,
      b_full_exer.reshape(1, 1), w_p3, b_p3.reshape(1, 1))


def _prep_edges(ei):
    pad = EPAD - EG
    tail = jnp.stack([jnp.zeros((pad,), jnp.int32),
                      jnp.full((pad,), N, jnp.int32)])
    return jnp.concatenate([ei.astype(jnp.int32), tail], axis=1)


def kernel(stu_id, exer_id, kn_emb, input_knowedge_ids, history,
           ei_sim, ei_pre, ei_exc,
           entity, e_disc_table, W_gcn, b_gcn,
           W_full_stu, b_full_stu, W_full_exer, b_full_exer, W_p3, b_p3):
    edges = jnp.stack([_prep_edges(ei_sim), _prep_edges(ei_pre),
                       _prep_edges(ei_exc)])

    e0c = _chunk_split(entity)
    deg = _sc_deg(edges)
    embs_c = [e0c]
    h_c = e0c
    for l in range(3):
        agg = _sc_propagate(h_c, edges)
        w3 = W_gcn[3 * l:3 * l + 3]
        bsum = jnp.sum(b_gcn[3 * l:3 * l + 3], axis=0, keepdims=True)
        h_c = _layer_matmul(agg, deg, w3, bsum)
        embs_c.append(h_c)

    all_chunks = [t for lc in embs_c for t in lc]
    hist_idx = (history.astype(jnp.int32) + K).reshape(-1)
    exer_idx = jnp.concatenate([exer_id.astype(jnp.int32) + K,
                                jnp.full((256 - B,), K, jnp.int32)])
    hist2, hist3, x0, x1, x2, x3 = _sc_final_gather(
        all_chunks, hist_idx, exer_idx)

    stu_emb = _pool(hist2, hist3)
    conc_stacks = [jnp.stack([c[:K] for c in lc]) for lc in embs_c]
    edisc_rows = e_disc_table[exer_id]
    return _final_stage(stu_emb, kn_emb, edisc_rows, conc_stacks,
                        (x0, x1, x2, x3),
                        W_full_stu, b_full_stu, W_full_exer, b_full_exer,
                        W_p3, b_p3)


# speedup vs baseline: 3.2923x; 1.1922x over previous
import functools

import jax
import jax.numpy as jnp
from jax import lax
from jax.experimental import pallas as pl
from jax.experimental.pallas import tpu as pltpu
from jax.experimental.pallas import tpu_sc as plsc

K = 512
N = 10000
EG = 150000
B = 128
H = 50

NC, NS, LANES = 2, 16, 16
NW = NC * NS
EB = 256
EPAD = 151552
NB = EPAD // NS // EB
NPAD = 10240
RPT = NPAD // NS
CHUNKS = 4
CW = 128
DEGW = 128

ROW_BLK = 400

@functools.cache
def _sc_mesh():
    return plsc.VectorSubcoreMesh(core_axis_name="c", subcore_axis_name="s",
                                  num_cores=NC, num_subcores=NS)


def _fill(ref, rows, width, val):
    def body(r, _):
        for k in range(width // LANES):
            ref[r, pl.ds(k * LANES, LANES)] = jnp.full((LANES,), val,
                                                       jnp.float32)
        return 0
    lax.fori_loop(0, rows, body, 0)


IG = 8


def _sc_propagate_body(*refs):
    (h0, h1, h2, h3, edges, agg_out,
     acc_sh, idxs, idxd, rows0, zero_v, gsem0) = refs
    tables = (h0, h1, h2, h3)
    rows = (rows0,)
    gsem = (gsem0,)
    cid = lax.axis_index("c")
    sid = lax.axis_index("s")
    r0 = pl.multiple_of(sid * RPT, 32)
    e0 = pl.multiple_of(sid * (EPAD // NS), 8)

    _fill(zero_v, 32, CW, 0.0)

    def zbody(j, _):
        pltpu.sync_copy(zero_v, acc_sh.at[pl.ds(r0 + j * 32, 32)])
        return 0

    lax.fori_loop(0, RPT // 32, zbody, 0)
    plsc.subcore_barrier()

    for g in range(3):
        for c in range(CHUNKS):
            owner = c // 2

            @pl.when(cid == owner)
            def _pass(g=g, c=c):
                tbl = tables[c]

                def body(b, _):
                    off = e0 + b * EB
                    pltpu.sync_copy(edges.at[g, 0, pl.ds(off, EB)], idxs)
                    pltpu.sync_copy(edges.at[g, 1, pl.ds(off, EB)], idxd)
                    pltpu.async_copy(tbl.at[idxs], rows[0], gsem[0]).wait()
                    pltpu.sync_copy(rows[0], acc_sh.at[idxd], add=True)
                    return 0

                lax.fori_loop(0, NB, body, 0)
                plsc.subcore_barrier()

                for j in range(RPT // 128):
                    rr = r0 + j * 128
                    fb = rows[0].at[pl.ds(0, 128)]
                    pltpu.sync_copy(acc_sh.at[pl.ds(rr, 128)], fb)
                    for z in range(4):
                        pltpu.sync_copy(zero_v,
                                        acc_sh.at[pl.ds(rr + z * 32, 32)])
                    pltpu.sync_copy(fb, agg_out.at[c, g, pl.ds(rr, 128)])
                plsc.subcore_barrier()


def _sc_propagate(h_chunks, edges):
    fn = pl.kernel(
        _sc_propagate_body,
        out_type=jax.ShapeDtypeStruct((CHUNKS, 3, NPAD, CW), jnp.float32),
        mesh=_sc_mesh(),
        scratch_types=[
            pltpu.VMEM_SHARED((NPAD, CW), jnp.float32),
            pltpu.VMEM((EB,), jnp.int32),
            pltpu.VMEM((EB,), jnp.int32),
            pltpu.VMEM((EB, CW), jnp.float32),
            pltpu.VMEM((32, CW), jnp.float32),
            pltpu.SemaphoreType.DMA,
        ],
    )
    return fn(*h_chunks, edges)


def _sc_deg_body(edges, deg_out, deg_sh, idx_d, ones_v, zero16, deg_buf,
                 sem):
    cid = lax.axis_index("c")
    sid = lax.axis_index("s")
    r0 = pl.multiple_of(sid * RPT, 32)
    e0 = pl.multiple_of(sid * (EPAD // NS), 8)

    _fill(ones_v, 128, DEGW, 1.0)
    _fill(zero16, 32, DEGW, 0.0)

    for g in range(3):
        owner = 0 if g < 2 else 1

        @pl.when(cid == owner)
        def _pass(g=g):
            def zbody(j, _):
                pltpu.sync_copy(zero16, deg_sh.at[pl.ds(r0 + j * 32, 32)])
                return 0

            lax.fori_loop(0, RPT // 32, zbody, 0)
            plsc.subcore_barrier()

            def body(b, _):
                off = e0 + b * 128
                pltpu.sync_copy(edges.at[g, 1, pl.ds(off, 128)], idx_d)
                pltpu.sync_copy(ones_v, deg_sh.at[idx_d], add=True)
                return 0

            lax.fori_loop(0, EPAD // NS // 128, body, 0)
            plsc.subcore_barrier()

            for j in range(RPT // 128):
                rr = r0 + j * 128
                pltpu.sync_copy(deg_sh.at[pl.ds(rr, 128)], deg_buf)
                pltpu.sync_copy(deg_buf, deg_out.at[g, pl.ds(rr, 128)])


def _sc_deg(edges):
    fn = pl.kernel(
        _sc_deg_body,
        out_type=jax.ShapeDtypeStruct((3, NPAD, DEGW), jnp.float32),
        mesh=_sc_mesh(),
        scratch_types=[
            pltpu.VMEM_SHARED((NPAD, DEGW), jnp.float32),
            pltpu.VMEM((128,), jnp.int32),
            pltpu.VMEM((128, DEGW), jnp.float32),
            pltpu.VMEM((32, DEGW), jnp.float32),
            pltpu.VMEM((128, DEGW), jnp.float32),
            pltpu.SemaphoreType.DMA,
        ],
    )
    return fn(edges)


def _sc_final_gather_body(*refs):
    (tabs, hist_idx, exer_idx,
     hist2, hist3, x0, x1, x2, x3,
     hidx128, hidx72, eidx, rows_v, rows72, rows8, sem) = (
        refs[:16], refs[16], refs[17],
        refs[18], refs[19], refs[20], refs[21], refs[22], refs[23],
        refs[24], refs[25], refs[26], refs[27], refs[28], refs[29],
        refs[30])
    cid = lax.axis_index("c")
    sid = lax.axis_index("s")
    wid = sid * NC + cid
    hb = wid * 200

    pltpu.sync_copy(hist_idx.at[pl.ds(hb, 128)], hidx128)
    pltpu.sync_copy(hist_idx.at[pl.ds(hb + 128, 72)], hidx72)
    for li, out in ((2, hist2), (3, hist3)):
        for c in range(CHUNKS):
            tab = tabs[li * CHUNKS + c]
            pltpu.async_copy(tab.at[hidx128], rows_v, sem).wait()
            pltpu.sync_copy(rows_v, out.at[c, pl.ds(hb, 128)])
            pltpu.async_copy(tab.at[hidx72], rows72, sem).wait()
            pltpu.sync_copy(rows72, out.at[c, pl.ds(hb + 128, 72)])

    eb = wid * 8
    pltpu.sync_copy(exer_idx.at[pl.ds(eb, 8)], eidx)
    for li, out in ((0, x0), (1, x1), (2, x2), (3, x3)):
        for c in range(CHUNKS):
            tab = tabs[li * CHUNKS + c]
            pltpu.async_copy(tab.at[eidx], rows8, sem).wait()
            pltpu.sync_copy(rows8, out.at[c, pl.ds(eb, 8)])


def _sc_final_gather(all_chunks, hist_idx, exer_idx):
    outs = (
        jax.ShapeDtypeStruct((CHUNKS, 6400, CW), jnp.float32),
        jax.ShapeDtypeStruct((CHUNKS, 6400, CW), jnp.float32),
        jax.ShapeDtypeStruct((CHUNKS, 256, CW), jnp.float32),
        jax.ShapeDtypeStruct((CHUNKS, 256, CW), jnp.float32),
        jax.ShapeDtypeStruct((CHUNKS, 256, CW), jnp.float32),
        jax.ShapeDtypeStruct((CHUNKS, 256, CW), jnp.float32),
    )
    fn = pl.kernel(
        _sc_final_gather_body,
        out_type=outs,
        mesh=_sc_mesh(),
        scratch_types=[
            pltpu.VMEM((128,), jnp.int32),
            pltpu.VMEM((72,), jnp.int32),
            pltpu.VMEM((8,), jnp.int32),
            pltpu.VMEM((128, CW), jnp.float32),
            pltpu.VMEM((72, CW), jnp.float32),
            pltpu.VMEM((8, CW), jnp.float32),
            pltpu.SemaphoreType.DMA,
        ],
    )
    return fn(*all_chunks, hist_idx, exer_idx)


def _chunk_body(x_ref, o0, o1, o2, o3):
    for c, o in enumerate((o0, o1, o2, o3)):
        o[...] = x_ref[:, c * CW:(c + 1) * CW]


def _chunk_split(x):
    grid = N // ROW_BLK
    return pl.pallas_call(
        _chunk_body,
        grid=(grid,),
        in_specs=[pl.BlockSpec((ROW_BLK, K), lambda i: (i, 0))],
        out_specs=[pl.BlockSpec((ROW_BLK, CW), lambda i: (i, 0))] * CHUNKS,
        out_shape=[jax.ShapeDtypeStruct((N, CW), jnp.float32)] * CHUNKS,
    )(x)


def _layer_mm_body(agg_ref, deg_ref, w_ref, bsum_ref, o0, o1, o2, o3):
    acc = jnp.zeros((ROW_BLK, K), jnp.float32)
    for g in range(3):
        rd = 1.0 / jnp.maximum(deg_ref[g, :, 0:1], 1.0)
        for c in range(CHUNKS):
            x = agg_ref[c, g] * rd
            acc = acc + jnp.dot(x, w_ref[g, c * CW:(c + 1) * CW, :],
                                preferred_element_type=jnp.float32)
    acc = acc + bsum_ref[...]
    for c, o in enumerate((o0, o1, o2, o3)):
        o[...] = acc[:, c * CW:(c + 1) * CW]


def _layer_matmul(agg, deg, w3, bsum):
    grid = N // ROW_BLK
    return pl.pallas_call(
        _layer_mm_body,
        grid=(grid,),
        in_specs=[
            pl.BlockSpec((CHUNKS, 3, ROW_BLK, CW), lambda i: (0, 0, i, 0)),
            pl.BlockSpec((3, ROW_BLK, DEGW), lambda i: (0, i, 0)),
            pl.BlockSpec((3, K, K), lambda i: (0, 0, 0)),
            pl.BlockSpec((1, K), lambda i: (0, 0)),
        ],
        out_specs=[pl.BlockSpec((ROW_BLK, CW), lambda i: (i, 0))] * CHUNKS,
        out_shape=[jax.ShapeDtypeStruct((N, CW), jnp.float32)] * CHUNKS,
    )(agg, deg, w3, bsum)


def _pool_body(h2_ref, h3_ref, out_ref):
    for c in range(CHUNKS):
        s = (h2_ref[c] + h3_ref[c]).reshape(16, H, CW)
        out_ref[:, c * CW:(c + 1) * CW] = jnp.sum(s, axis=1) * (0.5 / H)


def _pool(hist2, hist3):
    grid = B // 16
    return pl.pallas_call(
        _pool_body,
        grid=(grid,),
        in_specs=[pl.BlockSpec((CHUNKS, 16 * H, CW), lambda i: (0, i, 0))] * 2,
        out_specs=pl.BlockSpec((16, K), lambda i: (i, 0)),
        out_shape=jax.ShapeDtypeStruct((B, K), jnp.float32),
    )(hist2, hist3)


def _final_body(stu_ref, kn_ref, edisc_ref,
                c0, c1, c2, c3, x0, x1, x2, x3,
                wfs_ref, bfs_ref, wfe_ref, bfe_ref, wp3_ref, bp3_ref,
                out_ref):
    conc_c = [(c0[c] + c1[c] + c2[c] + c3[c]) * 0.25 for c in range(CHUNKS)]
    concept = jnp.concatenate(conc_c, axis=1)
    exer_c = [(x0[c] + x1[c] + x2[c] + x3[c])[0:B] * 0.25
              for c in range(CHUNKS)]
    exer = jnp.concatenate(exer_c, axis=1)

    wa = wfs_ref[0, :K]
    wb = wfs_ref[0, K:]
    wea = wfe_ref[0, :K]
    web = wfe_ref[0, K:]
    s = jnp.sum(stu_ref[...] * wa[None, :], axis=1, keepdims=True)
    t = jnp.sum(concept * wb[None, :], axis=1, keepdims=True)
    u = jnp.sum(exer * wea[None, :], axis=1, keepdims=True)
    v = jnp.sum(concept * web[None, :], axis=1, keepdims=True)
    prof = jax.nn.sigmoid(s + t[:, 0][None, :] + bfs_ref[0])
    kdiff = jax.nn.sigmoid(u + v[:, 0][None, :] + bfe_ref[0])
    edisc = jax.nn.sigmoid(edisc_ref[...]) * 10.0
    input_x = edisc * (prof - kdiff) * kn_ref[...]
    out_ref[...] = jax.nn.sigmoid(
        jnp.sum(input_x * wp3_ref[0][None, :], axis=1, keepdims=True)
        + bp3_ref[0])


def _final_stage(stu_emb, kn_emb, edisc_rows, conc_stacks, exer_rows,
                 w_full_stu, b_full_stu, w_full_exer, b_full_exer, w_p3, b_p3):
    return pl.pallas_call(
        _final_body,
        out_shape=jax.ShapeDtypeStruct((B, 1), jnp.float32),
    )(stu_emb, kn_emb, edisc_rows, *conc_stacks, *exer_rows,
      w_full_stu, b_full_stu.reshape(1, 1), w_full_exer,
      b_full_exer.reshape(1, 1), w_p3, b_p3.reshape(1, 1))


def _prep_edges(ei):
    pad = EPAD - EG
    tail = jnp.stack([jnp.zeros((pad,), jnp.int32),
                      jnp.full((pad,), N, jnp.int32)])
    return jnp.concatenate([ei.astype(jnp.int32), tail], axis=1)


def kernel(stu_id, exer_id, kn_emb, input_knowedge_ids, history,
           ei_sim, ei_pre, ei_exc,
           entity, e_disc_table, W_gcn, b_gcn,
           W_full_stu, b_full_stu, W_full_exer, b_full_exer, W_p3, b_p3):
    edges = jnp.stack([_prep_edges(ei_sim), _prep_edges(ei_pre),
                       _prep_edges(ei_exc)])

    e0c = _chunk_split(entity)
    deg = _sc_deg(edges)
    embs_c = [e0c]
    h_c = e0c
    for l in range(3):
        agg = _sc_propagate(h_c, edges)
        w3 = W_gcn[3 * l:3 * l + 3]
        bsum = jnp.sum(b_gcn[3 * l:3 * l + 3], axis=0, keepdims=True)
        h_c = _layer_matmul(agg, deg, w3, bsum)
        embs_c.append(h_c)

    all_chunks = [t for lc in embs_c for t in lc]
    hist_idx = (history.astype(jnp.int32) + K).reshape(-1)
    exer_idx = jnp.concatenate([exer_id.astype(jnp.int32) + K,
                                jnp.full((256 - B,), K, jnp.int32)])
    hist2, hist3, x0, x1, x2, x3 = _sc_final_gather(
        all_chunks, hist_idx, exer_idx)

    stu_emb = _pool(hist2, hist3)
    conc_stacks = [jnp.stack([c[:K] for c in lc]) for lc in embs_c]
    edisc_rows = e_disc_table[exer_id]
    return _final_stage(stu_emb, kn_emb, edisc_rows, conc_stacks,
                        (x0, x1, x2, x3),
                        W_full_stu, b_full_stu, W_full_exer, b_full_exer,
                        W_p3, b_p3)
